# baseline (speedup 1.0000x reference)
import math

import jax
import jax.numpy as jnp
import numpy as np
from jax.experimental import pallas as pl

_MAX_LEN = 288
_EMBED_DIM = 128


def _sin_enc(max_len, dim, period):
    position = np.arange(max_len, dtype=np.float32)[:, None]
    div_term = np.exp(np.arange(0, dim, 2, dtype=np.float32) * -(math.log(period) / dim))
    pe = np.zeros((max_len, dim), dtype=np.float32)
    pe[:, 0::2] = np.sin(position * div_term)
    pe[:, 1::2] = np.cos(position * div_term)
    return pe


def _build_pe_np():
    pe_standard = _sin_enc(_MAX_LEN, _EMBED_DIM // 2, 10000.0)
    pe_daily = _sin_enc(_MAX_LEN, _EMBED_DIM // 4, 288.0)
    pe_weekly = _sin_enc(_MAX_LEN, _EMBED_DIM // 4, 288.0 * 7)
    return np.concatenate([pe_standard, pe_daily, pe_weekly], axis=-1)


_PE = _build_pe_np()


def _add_kernel(x_ref, pe_ref, o_ref):
    o_ref[...] = x_ref[...] + pe_ref[...]


def kernel(x):
    B, N, T, D = x.shape
    pe = jnp.asarray(_PE[:T])
    rows = B * N
    x2 = x.reshape(rows, T, D)

    chunk = 32
    grid = rows // chunk

    out = pl.pallas_call(
        _add_kernel,
        grid=(grid,),
        in_specs=[
            pl.BlockSpec((chunk, T, D), lambda i: (i, 0, 0)),
            pl.BlockSpec((1, T, D), lambda i: (0, 0, 0)),
        ],
        out_specs=pl.BlockSpec((chunk, T, D), lambda i: (i, 0, 0)),
        out_shape=jax.ShapeDtypeStruct((rows, T, D), x.dtype),
    )(x2, pe[None])
    return out.reshape(B, N, T, D)


# speedup vs baseline: 1.0117x; 1.0117x over previous
import math

import jax
import jax.numpy as jnp
import numpy as np
from jax.experimental import pallas as pl

_MAX_LEN = 288
_EMBED_DIM = 128


def _sin_enc(max_len, dim, period):
    position = np.arange(max_len, dtype=np.float32)[:, None]
    div_term = np.exp(np.arange(0, dim, 2, dtype=np.float32) * -(math.log(period) / dim))
    pe = np.zeros((max_len, dim), dtype=np.float32)
    pe[:, 0::2] = np.sin(position * div_term)
    pe[:, 1::2] = np.cos(position * div_term)
    return pe


def _build_pe_np():
    pe_standard = _sin_enc(_MAX_LEN, _EMBED_DIM // 2, 10000.0)
    pe_daily = _sin_enc(_MAX_LEN, _EMBED_DIM // 4, 288.0)
    pe_weekly = _sin_enc(_MAX_LEN, _EMBED_DIM // 4, 288.0 * 7)
    return np.concatenate([pe_standard, pe_daily, pe_weekly], axis=-1)


_PE = _build_pe_np()


def _add_kernel(x_ref, pe_ref, o_ref):
    o_ref[...] = x_ref[...] + pe_ref[...]


def kernel(x):
    B, N, T, D = x.shape
    pe = jnp.asarray(_PE[:T])
    rows = B * N
    x2 = x.reshape(rows, T, D)

    chunk = 64
    grid = rows // chunk

    out = pl.pallas_call(
        _add_kernel,
        grid=(grid,),
        in_specs=[
            pl.BlockSpec((chunk, T, D), lambda i: (i, 0, 0)),
            pl.BlockSpec((1, T, D), lambda i: (0, 0, 0)),
        ],
        out_specs=pl.BlockSpec((chunk, T, D), lambda i: (i, 0, 0)),
        out_shape=jax.ShapeDtypeStruct((rows, T, D), x.dtype),
    )(x2, pe[None])
    return out.reshape(B, N, T, D)
